# baseline (speedup 1.0000x reference)
import functools

import jax
import jax.numpy as jnp
import numpy as np
from jax import lax
from jax.experimental import pallas as pl
from jax.experimental.pallas import tpu as pltpu
from jax.experimental.pallas import tpu_sc as plsc

N_DATA = 100000
BATCH = 16384
NC, NS, L = 2, 16, 16
NW = NC * NS
BPW = BATCH // NW
NT = BATCH // 128
CPAD = 4
SLAB = (BPW // 128) * CPAD * 128

_HALF_PI = np.float32(np.pi / 2)
_S3, _S5, _S7, _S9, _S11 = (np.float32(-1 / 6), np.float32(1 / 120),
                            np.float32(-1 / 5040), np.float32(1 / 362880),
                            np.float32(-1 / 39916800))
_C2, _C4, _C6, _C8, _C10 = (np.float32(-1 / 2), np.float32(1 / 24),
                            np.float32(-1 / 720), np.float32(1 / 40320),
                            np.float32(-1 / 3628800))


@functools.partial(
    pl.kernel,
    mesh=plsc.VectorSubcoreMesh(core_axis_name="c", subcore_axis_name="s"),
    out_type=jax.ShapeDtypeStruct((3 * CPAD * BATCH,), jnp.float32),
    scratch_types=[
        pltpu.VMEM((BPW,), jnp.int32),
        pltpu.VMEM((BPW, 8), jnp.float32),
        pltpu.VMEM((3 * SLAB,), jnp.float32),
        pltpu.SemaphoreType.DMA,
        pltpu.SemaphoreType.DMA,
    ],
    compiler_params=pltpu.CompilerParams(
        needs_layout_passes=False,
        use_tc_tiling_on_sc=False,
        skip_device_barrier=True,
        disable_bounds_checks=True,
        disable_semaphore_checks=True,
    ),
)
def _euler_rot_sc(idx_hbm, tab_hbm, out_hbm, idx_v, rows_v, out_v, gsem, osem):
    wid = lax.axis_index("s") * NC + lax.axis_index("c")
    base = wid * BPW

    pltpu.sync_copy(idx_hbm.at[pl.ds(base, BPW)], idx_v)

    pltpu.async_copy(tab_hbm.at[idx_v], rows_v, gsem).wait()

    lane = lax.iota(jnp.int32, L)
    zero = jnp.zeros((L,), jnp.int32)
    for i in range(BPW // L):
        s = i * L
        rows = lane + s
        c1 = plsc.load_gather(rows_v, [rows, zero])
        s1 = plsc.load_gather(rows_v, [rows, zero + 1])
        th = plsc.load_gather(rows_v, [rows, zero + 2])
        c2 = plsc.load_gather(rows_v, [rows, zero + 3])
        s2 = plsc.load_gather(rows_v, [rows, zero + 4])

        x = th - _HALF_PI
        x2 = x * x
        sinx = x * (1.0 + x2 * (_S3 + x2 * (_S5 + x2 * (_S7 + x2 * (_S9 + x2 * _S11)))))
        cosx = 1.0 + x2 * (_C2 + x2 * (_C4 + x2 * (_C6 + x2 * (_C8 + x2 * _C10))))
        ct = -sinx
        st = cosx

        a = c1 * ct
        b = s1 * ct
        entries = (
            (0, 0, a * c2 - s1 * s2),
            (0, 1, -(a * s2) - s1 * c2),
            (0, 2, c1 * st),
            (1, 0, b * c2 + c1 * s2),
            (1, 1, c1 * c2 - b * s2),
            (1, 2, s1 * st),
            (2, 0, -(st * c2)),
            (2, 1, st * s2),
            (2, 2, ct),
        )
        off = (i // 8) * (CPAD * 128) + (i % 8) * L
        for r, c, val in entries:
            out_v[pl.ds(r * SLAB + off + c * 128, L)] = val

    outs = [
        pltpu.async_copy(
            out_v.at[pl.ds(r * SLAB, SLAB)],
            out_hbm.at[pl.ds(r * (CPAD * BATCH) + wid * SLAB, SLAB)],
            osem,
        )
        for r in range(3)
    ]
    for o in outs:
        o.wait()


def kernel(idx, psi_cos, psi_sin, theta, phi_cos, phi_sin):
    tab = jnp.stack(
        [psi_cos, psi_sin, theta, phi_cos, phi_sin,
         psi_cos, psi_cos, psi_cos], axis=1)
    flat = _euler_rot_sc(idx.astype(jnp.int32), tab)
    grid = flat.reshape(3, NT, CPAD, 128)
    return grid.transpose(1, 3, 0, 2).reshape(BATCH, 3, CPAD)[:, :, :3]


# speedup vs baseline: 3.5474x; 3.5474x over previous
import functools

import jax
import jax.numpy as jnp
import numpy as np
from jax import lax
from jax.experimental import pallas as pl
from jax.experimental.pallas import tpu as pltpu
from jax.experimental.pallas import tpu_sc as plsc

N_DATA = 100000
BATCH = 16384
NC, NS, L = 1, 16, 16
NW = NC * NS
BPW = BATCH // NW
NT = BATCH // 128
CPAD = 4
SLAB = (BPW // 128) * CPAD * 128

_HALF_PI = np.float32(np.pi / 2)
_S3, _S5, _S7, _S9, _S11 = (np.float32(-1 / 6), np.float32(1 / 120),
                            np.float32(-1 / 5040), np.float32(1 / 362880),
                            np.float32(-1 / 39916800))
_C2, _C4, _C6, _C8, _C10 = (np.float32(-1 / 2), np.float32(1 / 24),
                            np.float32(-1 / 720), np.float32(1 / 40320),
                            np.float32(-1 / 3628800))


@functools.partial(
    pl.kernel,
    mesh=plsc.VectorSubcoreMesh(core_axis_name="c", subcore_axis_name="s", num_cores=1),
    out_type=jax.ShapeDtypeStruct((3 * CPAD * BATCH,), jnp.float32),
    scratch_types=[
        pltpu.VMEM((BPW,), jnp.int32),
        pltpu.VMEM((5 * BPW + 3 * SLAB,), jnp.float32),
        pltpu.SemaphoreType.DMA,
        pltpu.SemaphoreType.DMA,
    ],
    compiler_params=pltpu.CompilerParams(
        needs_layout_passes=False,
        skip_device_barrier=True,
        disable_bounds_checks=True,
        disable_semaphore_checks=True,
    ),
)
def _euler_rot_sc(idx_hbm, pc_hbm, ps_hbm, th_hbm, fc_hbm, fs_hbm, out_hbm,
                  idx_v, buf_v, gsem, osem):
    wid = lax.axis_index("s") * NC + lax.axis_index("c")
    base = wid * BPW

    pltpu.sync_copy(idx_hbm.at[pl.ds(base, BPW)], idx_v)

    copies = [
        pltpu.async_copy(t.at[idx_v], buf_v.at[pl.ds(k * BPW, BPW)], gsem)
        for k, t in enumerate((pc_hbm, ps_hbm, th_hbm, fc_hbm, fs_hbm))
    ]
    for c in copies:
        c.wait()

    OUT0 = 5 * BPW
    for i in range(BPW // L):
        s = i * L
        c1 = buf_v[pl.ds(s, L)]
        s1 = buf_v[pl.ds(BPW + s, L)]
        th = buf_v[pl.ds(2 * BPW + s, L)]
        c2 = buf_v[pl.ds(3 * BPW + s, L)]
        s2 = buf_v[pl.ds(4 * BPW + s, L)]

        x = th - _HALF_PI
        x2 = x * x
        sinx = x * (1.0 + x2 * (_S3 + x2 * (_S5 + x2 * (_S7 + x2 * (_S9 + x2 * _S11)))))
        cosx = 1.0 + x2 * (_C2 + x2 * (_C4 + x2 * (_C6 + x2 * (_C8 + x2 * _C10))))
        ct = -sinx
        st = cosx

        a = c1 * ct
        b = s1 * ct
        entries = (
            (0, 0, a * c2 - s1 * s2),
            (0, 1, -(a * s2) - s1 * c2),
            (0, 2, c1 * st),
            (1, 0, b * c2 + c1 * s2),
            (1, 1, c1 * c2 - b * s2),
            (1, 2, s1 * st),
            (2, 0, -(st * c2)),
            (2, 1, st * s2),
            (2, 2, ct),
        )
        off = (i // 8) * (CPAD * 128) + (i % 8) * L
        for r, c, val in entries:
            buf_v[pl.ds(OUT0 + r * SLAB + off + c * 128, L)] = val

    outs = [
        pltpu.async_copy(
            buf_v.at[pl.ds(OUT0 + r * SLAB, SLAB)],
            out_hbm.at[pl.ds(r * (CPAD * BATCH) + wid * SLAB, SLAB)],
            osem,
        )
        for r in range(3)
    ]
    for o in outs:
        o.wait()


def kernel(idx, psi_cos, psi_sin, theta, phi_cos, phi_sin):
    flat = _euler_rot_sc(idx.astype(jnp.int32), psi_cos, psi_sin, theta,
                         phi_cos, phi_sin)
    grid = flat.reshape(3, NT, CPAD, 128)
    return grid.transpose(1, 3, 0, 2).reshape(BATCH, 3, CPAD)[:, :, :3]


# speedup vs baseline: 4.0556x; 1.1432x over previous
import functools

import jax
import jax.numpy as jnp
import numpy as np
from jax import lax
from jax.experimental import pallas as pl
from jax.experimental.pallas import tpu as pltpu
from jax.experimental.pallas import tpu_sc as plsc

N_DATA = 100000
BATCH = 16384
NC, NS, L = 2, 16, 16
NW = NC * NS
BPW = BATCH // NW
NT = BATCH // 128
CPAD = 4
SLAB = (BPW // 128) * CPAD * 128

_HALF_PI = np.float32(np.pi / 2)
_S3, _S5, _S7, _S9, _S11 = (np.float32(-1 / 6), np.float32(1 / 120),
                            np.float32(-1 / 5040), np.float32(1 / 362880),
                            np.float32(-1 / 39916800))
_C2, _C4, _C6, _C8, _C10 = (np.float32(-1 / 2), np.float32(1 / 24),
                            np.float32(-1 / 720), np.float32(1 / 40320),
                            np.float32(-1 / 3628800))


@functools.partial(
    pl.kernel,
    mesh=plsc.VectorSubcoreMesh(core_axis_name="c", subcore_axis_name="s"),
    out_type=jax.ShapeDtypeStruct((3 * CPAD * BATCH,), jnp.float32),
    scratch_types=[
        pltpu.VMEM((BPW,), jnp.int32),
        pltpu.VMEM((5 * BPW + 3 * SLAB,), jnp.float32),
        pltpu.SemaphoreType.DMA,
        pltpu.SemaphoreType.DMA,
        pltpu.SemaphoreType.DMA,
    ],
    compiler_params=pltpu.CompilerParams(
        needs_layout_passes=False,
        skip_device_barrier=True,
        disable_bounds_checks=True,
        disable_semaphore_checks=True,
    ),
)
def _euler_rot_sc(idx_hbm, pc_hbm, ps_hbm, th_hbm, fc_hbm, fs_hbm, out_hbm,
                  idx_v, buf_v, gsem0, gsem1, osem):
    wid = lax.axis_index("s") * NC + lax.axis_index("c")
    base = wid * BPW
    tabs = (pc_hbm, ps_hbm, th_hbm, fc_hbm, fs_hbm)
    H = BPW // 2
    OUT0 = 5 * BPW

    pltpu.sync_copy(idx_hbm.at[pl.ds(base, BPW)], idx_v)

    gath0 = [
        pltpu.async_copy(t.at[idx_v.at[pl.ds(0, H)]],
                         buf_v.at[pl.ds(k * BPW, H)], gsem0)
        for k, t in enumerate(tabs)
    ]
    gath1 = [
        pltpu.async_copy(t.at[idx_v.at[pl.ds(H, H)]],
                         buf_v.at[pl.ds(k * BPW + H, H)], gsem1)
        for k, t in enumerate(tabs)
    ]

    def chunk(i):
        s = i * L
        c1 = buf_v[pl.ds(s, L)]
        s1 = buf_v[pl.ds(BPW + s, L)]
        th = buf_v[pl.ds(2 * BPW + s, L)]
        c2 = buf_v[pl.ds(3 * BPW + s, L)]
        s2 = buf_v[pl.ds(4 * BPW + s, L)]

        x = th - _HALF_PI
        x2 = x * x
        sinx = x * (1.0 + x2 * (_S3 + x2 * (_S5 + x2 * (_S7 + x2 * (_S9 + x2 * _S11)))))
        cosx = 1.0 + x2 * (_C2 + x2 * (_C4 + x2 * (_C6 + x2 * (_C8 + x2 * _C10))))
        ct = -sinx
        st = cosx

        a = c1 * ct
        b = s1 * ct
        entries = (
            (0, 0, a * c2 - s1 * s2),
            (0, 1, -(a * s2) - s1 * c2),
            (0, 2, c1 * st),
            (1, 0, b * c2 + c1 * s2),
            (1, 1, c1 * c2 - b * s2),
            (1, 2, s1 * st),
            (2, 0, -(st * c2)),
            (2, 1, st * s2),
            (2, 2, ct),
        )
        off = (i // 8) * (CPAD * 128) + (i % 8) * L
        for r, c, val in entries:
            buf_v[pl.ds(OUT0 + r * SLAB + off + c * 128, L)] = val

    SUB = CPAD * 128
    outs = []

    def flush(t_local):
        for r in range(3):
            src = OUT0 + r * SLAB + t_local * SUB
            dst = r * (CPAD * BATCH) + wid * SLAB + t_local * SUB
            outs.append(pltpu.async_copy(
                buf_v.at[pl.ds(src, SUB)], out_hbm.at[pl.ds(dst, SUB)], osem))

    for c in gath0:
        c.wait()
    for i in range(H // L):
        chunk(i)
        if i % 8 == 7:
            flush(i // 8)
    for c in gath1:
        c.wait()
    for i in range(H // L, BPW // L):
        chunk(i)
        if i % 8 == 7:
            flush(i // 8)
    for o in outs:
        o.wait()


def kernel(idx, psi_cos, psi_sin, theta, phi_cos, phi_sin):
    flat = _euler_rot_sc(idx.astype(jnp.int32), psi_cos, psi_sin, theta,
                         phi_cos, phi_sin)
    grid = flat.reshape(3, NT, CPAD, 128)
    return grid.transpose(1, 3, 0, 2).reshape(BATCH, 3, CPAD)[:, :, :3]


# speedup vs baseline: 4.0960x; 1.0100x over previous
import functools

import jax
import jax.numpy as jnp
import numpy as np
from jax import lax
from jax.experimental import pallas as pl
from jax.experimental.pallas import tpu as pltpu
from jax.experimental.pallas import tpu_sc as plsc

N_DATA = 100000
BATCH = 16384
NC, NS, L = 2, 16, 16
NW = NC * NS
BPW = BATCH // NW
NT = BATCH // 128
CPAD = 4
SLAB = (BPW // 128) * CPAD * 128

_HALF_PI = np.float32(np.pi / 2)
_S3, _S5, _S7, _S9, _S11 = (np.float32(-1 / 6), np.float32(1 / 120),
                            np.float32(-1 / 5040), np.float32(1 / 362880),
                            np.float32(-1 / 39916800))
_C2, _C4, _C6, _C8, _C10 = (np.float32(-1 / 2), np.float32(1 / 24),
                            np.float32(-1 / 720), np.float32(1 / 40320),
                            np.float32(-1 / 3628800))


@functools.partial(
    pl.kernel,
    mesh=plsc.VectorSubcoreMesh(core_axis_name="c", subcore_axis_name="s"),
    out_type=jax.ShapeDtypeStruct((3 * CPAD * BATCH,), jnp.float32),
    scratch_types=[
        pltpu.VMEM((BPW,), jnp.int32),
        pltpu.VMEM((5 * BPW + 3 * SLAB,), jnp.float32),
        pltpu.SemaphoreType.DMA,
        pltpu.SemaphoreType.DMA,
        pltpu.SemaphoreType.DMA,
        pltpu.SemaphoreType.DMA,
        pltpu.SemaphoreType.DMA,
    ],
    compiler_params=pltpu.CompilerParams(
        needs_layout_passes=False,
        skip_device_barrier=True,
        disable_bounds_checks=True,
        disable_semaphore_checks=True,
    ),
)
def _euler_rot_sc(idx_hbm, pc_hbm, ps_hbm, th_hbm, fc_hbm, fs_hbm, out_hbm,
                  idx_v, buf_v, isem0, isem1, gsem0, gsem1, osem):
    wid = lax.axis_index("s") * NC + lax.axis_index("c")
    base = wid * BPW
    tabs = (pc_hbm, ps_hbm, th_hbm, fc_hbm, fs_hbm)
    H = BPW // 2
    OUT0 = 5 * BPW

    idx0 = pltpu.async_copy(idx_hbm.at[pl.ds(base, H)],
                            idx_v.at[pl.ds(0, H)], isem0)
    idx1 = pltpu.async_copy(idx_hbm.at[pl.ds(base + H, H)],
                            idx_v.at[pl.ds(H, H)], isem1)

    idx0.wait()
    gath0 = [
        pltpu.async_copy(t.at[idx_v.at[pl.ds(0, H)]],
                         buf_v.at[pl.ds(k * BPW, H)], gsem0)
        for k, t in enumerate(tabs)
    ]
    idx1.wait()
    gath1 = [
        pltpu.async_copy(t.at[idx_v.at[pl.ds(H, H)]],
                         buf_v.at[pl.ds(k * BPW + H, H)], gsem1)
        for k, t in enumerate(tabs)
    ]

    def chunk(i):
        s = i * L
        c1 = buf_v[pl.ds(s, L)]
        s1 = buf_v[pl.ds(BPW + s, L)]
        th = buf_v[pl.ds(2 * BPW + s, L)]
        c2 = buf_v[pl.ds(3 * BPW + s, L)]
        s2 = buf_v[pl.ds(4 * BPW + s, L)]

        x = th - _HALF_PI
        x2 = x * x
        sinx = x * (1.0 + x2 * (_S3 + x2 * (_S5 + x2 * (_S7 + x2 * (_S9 + x2 * _S11)))))
        cosx = 1.0 + x2 * (_C2 + x2 * (_C4 + x2 * (_C6 + x2 * (_C8 + x2 * _C10))))
        ct = -sinx
        st = cosx

        a = c1 * ct
        b = s1 * ct
        entries = (
            (0, 0, a * c2 - s1 * s2),
            (0, 1, -(a * s2) - s1 * c2),
            (0, 2, c1 * st),
            (1, 0, b * c2 + c1 * s2),
            (1, 1, c1 * c2 - b * s2),
            (1, 2, s1 * st),
            (2, 0, -(st * c2)),
            (2, 1, st * s2),
            (2, 2, ct),
        )
        off = (i // 8) * (CPAD * 128) + (i % 8) * L
        for r, c, val in entries:
            buf_v[pl.ds(OUT0 + r * SLAB + off + c * 128, L)] = val

    SUB = CPAD * 128
    outs = []

    def flush(t_local):
        for r in range(3):
            src = OUT0 + r * SLAB + t_local * SUB
            dst = r * (CPAD * BATCH) + wid * SLAB + t_local * SUB
            outs.append(pltpu.async_copy(
                buf_v.at[pl.ds(src, SUB)], out_hbm.at[pl.ds(dst, SUB)], osem))

    for c in gath0:
        c.wait()
    for i in range(H // L):
        chunk(i)
        if i % 8 == 7:
            flush(i // 8)
    for c in gath1:
        c.wait()
    for i in range(H // L, BPW // L):
        chunk(i)
        if i % 8 == 7:
            flush(i // 8)
    for o in outs:
        o.wait()


def kernel(idx, psi_cos, psi_sin, theta, phi_cos, phi_sin):
    flat = _euler_rot_sc(idx.astype(jnp.int32), psi_cos, psi_sin, theta,
                         phi_cos, phi_sin)
    grid = flat.reshape(3, NT, CPAD, 128)
    return grid.transpose(1, 3, 0, 2).reshape(BATCH, 3, CPAD)[:, :, :3]
